# Initial kernel scaffold; baseline (speedup 1.0000x reference)
import jax
import jax.numpy as jnp
from jax.experimental import pallas as pl


def kernel(x, facility_table, time_table):
    raise NotImplementedError("write your pallas kernel here")


# speedup vs baseline: 5.4440x; 5.4440x over previous
import functools

import jax
import jax.numpy as jnp
from jax import lax
from jax.experimental import pallas as pl
from jax.experimental.pallas import tpu as pltpu
from jax.experimental.pallas import tpu_sc as plsc

_NUM_CORES = 2
_NUM_SUBCORES = 16
_NUM_WORKERS = _NUM_CORES * _NUM_SUBCORES
_LANES = 16

_CHUNK = 128


@functools.partial(jax.jit, static_argnames=("n_rows", "d"))
def _sc_lookup(t_all, f_all, facility_table, time_table, n_rows, d):
    rows_per_w = n_rows // _NUM_WORKERS
    n_chunks = rows_per_w // _CHUNK

    mesh = plsc.VectorSubcoreMesh(
        core_axis_name="c", subcore_axis_name="s",
        num_cores=_NUM_CORES, num_subcores=_NUM_SUBCORES)

    @functools.partial(
        pl.kernel,
        out_type=jax.ShapeDtypeStruct((n_rows, d), jnp.float32),
        mesh=mesh,
        scratch_types=[
            pltpu.VMEM((_CHUNK,), jnp.int32),
            pltpu.VMEM((_CHUNK,), jnp.int32),
            pltpu.VMEM((_CHUNK,), jnp.int32),
            pltpu.VMEM((_CHUNK, d), jnp.float32),
            pltpu.VMEM((_CHUNK, d), jnp.float32),
            pltpu.SemaphoreType.DMA,
            pltpu.SemaphoreType.DMA,
        ],
    )
    def k(t_hbm, f_hbm, fac_hbm, time_hbm, out_hbm,
          t_v, f_v, ti_v, rows_t, rows_f, sem_t, sem_f):
        wid = lax.axis_index("s") * _NUM_CORES + lax.axis_index("c")
        base0 = wid * rows_per_w

        def chunk_body(g, carry):
            base = base0 + g * _CHUNK
            pltpu.sync_copy(t_hbm.at[pl.ds(base, _CHUNK)], t_v)
            pltpu.sync_copy(f_hbm.at[pl.ds(base, _CHUNK)], f_v)

            def sel_body(i, c):
                off = i * _LANES
                fv = f_v[pl.ds(off, _LANES)]
                tv = t_v[pl.ds(off, _LANES)]
                zero = jnp.zeros((_LANES,), jnp.int32)
                ti_v[pl.ds(off, _LANES)] = jnp.where(fv == 0, zero, tv)
                return c
            lax.fori_loop(0, _CHUNK // _LANES, sel_body, 0)

            cp_t = pltpu.async_copy(time_hbm.at[ti_v], rows_t, sem_t)
            cp_f = pltpu.async_copy(fac_hbm.at[f_v], rows_f, sem_f)
            cp_t.wait()
            cp_f.wait()

            def add_body(i, c):
                for j in range(d // _LANES):
                    v = rows_f[i, pl.ds(j * _LANES, _LANES)]
                    plsc.addupdate(rows_t.at[i, pl.ds(j * _LANES, _LANES)], v)
                return c
            lax.fori_loop(0, _CHUNK, add_body, 0)

            pltpu.sync_copy(rows_t, out_hbm.at[pl.ds(base, _CHUNK)])
            return carry

        lax.fori_loop(0, n_chunks, chunk_body, 0)

    return k(t_all, f_all, facility_table, time_table)


def kernel(x, facility_table, time_table):
    b, l, _ = x.shape
    d = facility_table.shape[1]
    n_rows = b * l
    t_all = x[:, :, 0].reshape(n_rows)
    f_all = x[:, :, 1].reshape(n_rows)
    out = _sc_lookup(t_all, f_all, facility_table, time_table, n_rows, d)
    return out.reshape(b, l, d)


# speedup vs baseline: 6.1571x; 1.1310x over previous
import functools

import jax
import jax.numpy as jnp
from jax import lax
from jax.experimental import pallas as pl
from jax.experimental.pallas import tpu as pltpu
from jax.experimental.pallas import tpu_sc as plsc

_NUM_CORES = 2
_NUM_SUBCORES = 16
_NUM_WORKERS = _NUM_CORES * _NUM_SUBCORES
_LANES = 16

_CHUNK = 128


@functools.partial(jax.jit, static_argnames=("n_rows", "d"))
def _sc_lookup(t_all, f_all, facility_table, time_table, n_rows, d):
    rows_per_w = n_rows // _NUM_WORKERS
    n_chunks = rows_per_w // _CHUNK

    mesh = plsc.VectorSubcoreMesh(
        core_axis_name="c", subcore_axis_name="s",
        num_cores=_NUM_CORES, num_subcores=_NUM_SUBCORES)

    @functools.partial(
        pl.kernel,
        out_type=jax.ShapeDtypeStruct((n_rows, d), jnp.float32),
        mesh=mesh,
        scratch_types=[
            pltpu.VMEM((_CHUNK,), jnp.int32),
            pltpu.VMEM((_CHUNK,), jnp.int32),
            pltpu.VMEM((_CHUNK,), jnp.int32),
            pltpu.VMEM((_CHUNK,), jnp.int32),
            pltpu.VMEM((_CHUNK,), jnp.int32),
            pltpu.VMEM((_CHUNK,), jnp.int32),
            pltpu.VMEM((_CHUNK, d), jnp.float32),
            pltpu.VMEM((_CHUNK, d), jnp.float32),
            pltpu.VMEM((_CHUNK, d), jnp.float32),
            pltpu.VMEM((_CHUNK, d), jnp.float32),
            pltpu.SemaphoreType.DMA,
            pltpu.SemaphoreType.DMA,
            pltpu.SemaphoreType.DMA,
            pltpu.SemaphoreType.DMA,
            pltpu.SemaphoreType.DMA,
            pltpu.SemaphoreType.DMA,
        ],
    )
    def k(t_hbm, f_hbm, fac_hbm, time_hbm, out_hbm, *scr):
        t_v, f_v, ti_v = scr[0:2], scr[2:4], scr[4:6]
        rows_t, rows_f = scr[6:8], scr[8:10]
        sem_in, sem_g, sem_out = scr[10:12], scr[12:14], scr[14:16]

        wid = lax.axis_index("s") * _NUM_CORES + lax.axis_index("c")
        base0 = wid * rows_per_w

        def in_descs(g, b):
            base = base0 + g * _CHUNK
            return (
                pltpu.make_async_copy(
                    t_hbm.at[pl.ds(base, _CHUNK)], t_v[b], sem_in[b]),
                pltpu.make_async_copy(
                    f_hbm.at[pl.ds(base, _CHUNK)], f_v[b], sem_in[b]),
            )

        def gather_descs(b):
            return (
                pltpu.make_async_copy(time_hbm.at[ti_v[b]], rows_t[b], sem_g[b]),
                pltpu.make_async_copy(fac_hbm.at[f_v[b]], rows_f[b], sem_g[b]),
            )

        def out_desc(g, b):
            base = base0 + g * _CHUNK
            return pltpu.make_async_copy(
                rows_t[b], out_hbm.at[pl.ds(base, _CHUNK)], sem_out[b])

        def do_select(b):
            def body(i, c):
                off = i * _LANES
                fv = f_v[b][pl.ds(off, _LANES)]
                tv = t_v[b][pl.ds(off, _LANES)]
                zero = jnp.zeros((_LANES,), jnp.int32)
                ti_v[b][pl.ds(off, _LANES)] = jnp.where(fv == 0, zero, tv)
                return c
            lax.fori_loop(0, _CHUNK // _LANES, body, 0)

        def do_add(b):
            def body(i, c):
                for j in range(d // _LANES):
                    sl = pl.ds(j * _LANES, _LANES)
                    plsc.addupdate(rows_t[b].at[i, sl], rows_f[b][i, sl])
                return c
            lax.fori_loop(0, _CHUNK, body, 0)

        def step(g, b, *, first=False, has_next=True, start_next_in=True):
            for cd in gather_descs(b):
                cd.wait()
            if has_next:
                for cd in in_descs(g + 1, 1 - b):
                    cd.wait()
                do_select(1 - b)
                if not first:
                    out_desc(g - 1, 1 - b).wait()
                for cd in gather_descs(1 - b):
                    cd.start()
                if start_next_in:
                    for cd in in_descs(g + 2, b):
                        cd.start()
            do_add(b)
            out_desc(g, b).start()

        for cd in in_descs(0, 0):
            cd.start()
        for cd in in_descs(0, 0):
            cd.wait()
        do_select(0)
        for cd in gather_descs(0):
            cd.start()
        for cd in in_descs(1, 1):
            cd.start()

        step(0, 0, first=True)
        step(1, 1)

        def pair(k2, c):
            g = 2 * k2
            step(g, 0)
            step(g + 1, 1)
            return c
        lax.fori_loop(1, (n_chunks - 2) // 2, pair, 0)

        step(n_chunks - 2, 0, start_next_in=False)
        step(n_chunks - 1, 1, has_next=False)
        out_desc(n_chunks - 2, 0).wait()
        out_desc(n_chunks - 1, 1).wait()

    return k(t_all, f_all, facility_table, time_table)


def kernel(x, facility_table, time_table):
    b, l, _ = x.shape
    d = facility_table.shape[1]
    n_rows = b * l
    t_all = x[:, :, 0].reshape(n_rows)
    f_all = x[:, :, 1].reshape(n_rows)
    out = _sc_lookup(t_all, f_all, facility_table, time_table, n_rows, d)
    return out.reshape(b, l, d)
